# baseline (speedup 1.0000x reference)
import functools

import numpy as np
import jax
import jax.numpy as jnp
from jax import lax
from jax.experimental import pallas as pl
from jax.experimental.pallas import tpu as pltpu
from jax.experimental.pallas import tpu_sc as plsc

B, AT, NBR, NBRT = 4, 128, 32, 96
F = 128
NGD = 25
NGT = 25
NTH = 10
ZETA = 8.0
CUTOFF = 6.0
NINT = 3
MAXZ = 100

BA = B * AT
RTB = AT * NBRT
RDB = AT * NBR
NCH = 1
RTC = RTB // NCH
FTW = NGT * NTH

_LOG2 = float(np.log(2.0))

_offs = np.linspace(0.001, CUTOFF - 0.5, NGT)
_W2 = float(_offs[1] - _offs[0]) ** 2
_OFFCOL = _offs.reshape(NGT, 1).astype(np.float32)
_theta = np.linspace(0.0, np.pi, NTH)
_CTCOL = np.cos(_theta).reshape(NTH, 1).astype(np.float32)
_STCOL = np.sin(_theta).reshape(NTH, 1).astype(np.float32)
_EGT = np.zeros((FTW, NGT), np.float32)
_ETT = np.zeros((FTW, NTH), np.float32)
for _g in range(NGT):
    for _t in range(NTH):
        _EGT[_g * NTH + _t, _g] = 1.0
        _ETT[_g * NTH + _t, _t] = 1.0
_REPT = np.kron(np.eye(AT, dtype=np.float32), np.ones((1, NBRT), np.float32))
_REPD = np.kron(np.eye(AT, dtype=np.float32), np.ones((1, NBR), np.float32))
_SEGT = _REPT.T.copy()
_SEGD = _REPD.T.copy()


def _ssp(x):
    return jnp.maximum(x, 0.0) + jnp.log(1.0 + jnp.exp(-jnp.abs(x))) - _LOG2


def _cutoff(r):
    return 0.5 * (jnp.cos(r * (np.pi / CUTOFF)) + 1.0) * (r < CUTOFF).astype(r.dtype)


def _dot(a, b):
    return jnp.dot(a, b, preferred_element_type=jnp.float32)


_SC_NC = 2
_SC_NS = 16
_NW = _SC_NC * _SC_NS
_EPW = BA // _NW


def _embed_sc_body(atn_hbm, emb_hbm, out_hbm, idx_v, rows_v, sem):
    wid = lax.axis_index("s") * _SC_NC + lax.axis_index("c")
    base = wid * _EPW
    pltpu.sync_copy(atn_hbm.at[pl.ds(base, _EPW)], idx_v)
    pltpu.async_copy(emb_hbm.at[idx_v], rows_v, sem).wait()
    pltpu.sync_copy(rows_v, out_hbm.at[pl.ds(base, _EPW)])


def _embed_sc(atn_flat, emb):
    k = functools.partial(
        pl.kernel,
        mesh=plsc.VectorSubcoreMesh(core_axis_name="c", subcore_axis_name="s"),
        out_type=jax.ShapeDtypeStruct((BA, F), jnp.float32),
        scratch_types=[
            pltpu.VMEM((_EPW,), jnp.int32),
            pltpu.VMEM((_EPW, F), jnp.float32),
            pltpu.SemaphoreType.DMA,
        ],
    )(_embed_sc_body)
    return k(atn_flat, emb)


def _fused_body(x0_ref, pos_ref, nbrd_ref, nbrj_ref, nbrk_ref,
                nmask_ref, tmask_ref,
                offc_ref, ctc_ref, stc_ref, egt_ref, ett_ref,
                rept_ref, repd_ref, segt_ref, segd_ref,
                fdw1_ref, fdb1_ref, fdw2_ref, fdb2_ref,
                ftw1_ref, ftb1_ref, ftw2_ref, ftb2_ref,
                in2f_ref, f2ow_ref, f2ob_ref, dw_ref, db_ref,
                xo_ref):
    bf16 = jnp.bfloat16
    f32 = jnp.float32
    pos3 = pos_ref[0]
    pos_t = jnp.concatenate(
        [pos3, jnp.zeros((AT, 5), f32)], axis=1).T
    pos_hi = pos_t.astype(bf16)
    pos_lo = (pos_t - pos_hi.astype(f32)).astype(bf16)
    pos_hl = jnp.concatenate([pos_hi, pos_lo], axis=0)
    offc = offc_ref[...]

    io_t = jax.lax.broadcasted_iota(jnp.int32, (AT, RTB), 0)

    def onehot_b(idx_row, width):
        return (io_t[:, :width] == idx_row).astype(bf16)

    def gather_pos(oh_b):
        g = _dot(pos_hl, oh_b)
        return g[:8, :] + g[8:, :]

    bias_cols = jnp.concatenate(
        [fdb1_ref[...], fdb2_ref[...], ftb1_ref[...], ftb2_ref[...],
         f2ob_ref[...], db_ref[...]], axis=0).T

    def bcol(j):
        return bias_cols[:, j:j + 1]

    ohd_b = onehot_b(nbrd_ref[0], RDB)
    pj_d = gather_pos(ohd_b)
    pi_d = gather_pos(repd_ref[...])
    vd = pj_d - pi_d
    rdst = jnp.sqrt(jnp.sum(vd * vd, axis=0, keepdims=True) + 1e-9)
    sdd = rdst - offc
    fd_b = jnp.exp((-0.5 / _W2) * sdd * sdd).astype(bf16)
    cdf = _cutoff(rdst) * nmask_ref[0]

    ohj_b = onehot_b(nbrj_ref[0], RTB)
    ohk_b = onehot_b(nbrk_ref[0], RTB)
    pi_t = gather_pos(rept_ref[...])
    vij = gather_pos(ohj_b) - pi_t
    vik = gather_pos(ohk_b) - pi_t
    rij = jnp.sqrt(jnp.sum(vij * vij, axis=0, keepdims=True) + 1e-9)
    rik = jnp.sqrt(jnp.sum(vik * vik, axis=0, keepdims=True) + 1e-9)
    cost = jnp.sum(vij * vik, axis=0, keepdims=True) / (rij * rik)
    cost = jnp.clip(cost, -1.0 + 1e-6, 1.0 - 1e-6)
    sint = jnp.sqrt(1.0 - cost * cost)
    sij = rij - offc
    sik = rik - offc
    gr_b = jnp.exp((-0.5 / _W2) * (sij * sij + sik * sik)).astype(bf16)
    base = 1.0 + ctc_ref[...] * cost + stc_ref[...] * sint
    b2 = base * base
    b4 = b2 * b2
    ang_b = ((b4 * b4) * (2.0 ** (1.0 - ZETA))).astype(bf16)
    rboth = jnp.concatenate([rij, rik], axis=0)
    cutb = _cutoff(rboth)
    ctf = cutb[0:1, :] * cutb[1:2, :] * tmask_ref[0]

    egt_b = egt_ref[...].astype(bf16)
    ett_b = ett_ref[...].astype(bf16)
    segt_b = segt_ref[...]
    segd_b = segd_ref[...]

    x_t = x0_ref[...].T
    for i in range(NINT):
        fdw1t = fdw1_ref[i].T.astype(bf16)
        fdw2t = fdw2_ref[i].T.astype(bf16)
        ftw1t = ftw1_ref[i].T.astype(bf16)
        ftw2t = ftw2_ref[i].T.astype(bf16)
        in2ft = in2f_ref[i].T
        f2owt = f2ow_ref[i].T
        dwt = dw_ref[i].T
        y_t = _dot(in2ft, x_t)
        y_b = y_t.astype(bf16)
        hd = _ssp(_dot(fdw1t, fd_b) + bcol(i))
        wd = (_dot(fdw2t, hd.astype(bf16)) + bcol(NINT + i)) * cdf
        prod_d = (_dot(y_b, ohd_b) * wd).astype(bf16)
        agg = _dot(prod_d, segd_b)
        for c in range(NCH):
            lo, hi = c * RTC, (c + 1) * RTC
            feat = (_dot(egt_b, gr_b[:, lo:hi])
                    * _dot(ett_b, ang_b[:, lo:hi])).astype(bf16)
            ht = _ssp(_dot(ftw1t, feat) + bcol(2 * NINT + i))
            wt = (_dot(ftw2t, ht.astype(bf16))
                  + bcol(3 * NINT + i)) * ctf[:, lo:hi]
            prod_t = (_dot(y_b, ohj_b[:, lo:hi]) * _dot(y_b, ohk_b[:, lo:hi])
                      * wt).astype(bf16)
            agg = agg + _dot(prod_t, segt_b[lo:hi, :])
        v = _ssp(_dot(f2owt, agg) + bcol(4 * NINT + i))
        x_t = x_t + _dot(dwt, v) + bcol(5 * NINT + i)
    xo_ref[0] = x_t.T


def kernel(atomic_numbers, positions, neighbors, neighbor_mask, neighbors_j,
           neighbors_k, triple_mask, emb, fd_W1, fd_b1, fd_W2, fd_b2,
           ft_W1, ft_b1, ft_W2, ft_b2, in2f_W, f2out_W, f2out_b,
           dense_W, dense_b):
    f32 = jnp.float32
    bf16 = jnp.bfloat16
    nbrd_b = neighbors.astype(jnp.int32).reshape(B, 1, RDB)
    nbrj_b = neighbors_j.astype(jnp.int32).reshape(B, 1, RTB)
    nbrk_b = neighbors_k.astype(jnp.int32).reshape(B, 1, RTB)
    nmask_b = neighbor_mask.astype(f32).reshape(B, 1, RDB)
    tmask_b = triple_mask.astype(f32).reshape(B, 1, RTB)

    x0 = _embed_sc(atomic_numbers.astype(jnp.int32).reshape(BA),
                   emb.astype(f32))

    whole = lambda *shape: pl.BlockSpec(shape, lambda g: tuple(0 for _ in shape))
    perb = lambda *shape: pl.BlockSpec((1,) + shape, lambda g: (g,) + tuple(
        0 for _ in shape))

    out = pl.pallas_call(
        _fused_body,
        grid=(B,),
        in_specs=[
            pl.BlockSpec((AT, F), lambda g: (g, 0)),
            perb(AT, 3),
            perb(1, RDB), perb(1, RTB), perb(1, RTB),
            perb(1, RDB), perb(1, RTB),
            whole(NGT, 1), whole(NTH, 1), whole(NTH, 1),
            whole(FTW, NGT), whole(FTW, NTH),
            whole(AT, RTB), whole(AT, RDB),
            whole(RTB, AT), whole(RDB, AT),
            whole(NINT, NGD, F), whole(NINT, F),
            whole(NINT, F, F), whole(NINT, F),
            whole(NINT, FTW, F), whole(NINT, F),
            whole(NINT, F, F), whole(NINT, F),
            whole(NINT, F, F), whole(NINT, F, F), whole(NINT, F),
            whole(NINT, F, F), whole(NINT, F),
        ],
        out_specs=pl.BlockSpec((1, AT, F), lambda g: (g, 0, 0)),
        out_shape=jax.ShapeDtypeStruct((B, AT, F), f32),
    )(x0, positions, nbrd_b, nbrj_b, nbrk_b, nmask_b, tmask_b,
      jnp.asarray(_OFFCOL), jnp.asarray(_CTCOL), jnp.asarray(_STCOL),
      jnp.asarray(_EGT), jnp.asarray(_ETT),
      jnp.asarray(_REPT).astype(bf16), jnp.asarray(_REPD).astype(bf16),
      jnp.asarray(_SEGT).astype(bf16), jnp.asarray(_SEGD).astype(bf16),
      fd_W1, fd_b1, fd_W2, fd_b2, ft_W1, ft_b1, ft_W2, ft_b2,
      in2f_W, f2out_W, f2out_b, dense_W, dense_b)
    return out


# speedup vs baseline: 1.0131x; 1.0131x over previous
import functools

import numpy as np
import jax
import jax.numpy as jnp
from jax import lax
from jax.experimental import pallas as pl
from jax.experimental.pallas import tpu as pltpu
from jax.experimental.pallas import tpu_sc as plsc

B, AT, NBR, NBRT = 4, 128, 32, 96
F = 128
NGD = 25
NGT = 25
NTH = 10
ZETA = 8.0
CUTOFF = 6.0
NINT = 3
MAXZ = 100

BA = B * AT
RTB = AT * NBRT
RDB = AT * NBR
NCH = 2
RTC = RTB // NCH
FTW = NGT * NTH

_LOG2 = float(np.log(2.0))

_offs = np.linspace(0.001, CUTOFF - 0.5, NGT)
_W2 = float(_offs[1] - _offs[0]) ** 2
_OFFCOL = _offs.reshape(NGT, 1).astype(np.float32)
_theta = np.linspace(0.0, np.pi, NTH)
_CTCOL = np.cos(_theta).reshape(NTH, 1).astype(np.float32)
_STCOL = np.sin(_theta).reshape(NTH, 1).astype(np.float32)
_EGT = np.zeros((FTW, NGT), np.float32)
_ETT = np.zeros((FTW, NTH), np.float32)
for _g in range(NGT):
    for _t in range(NTH):
        _EGT[_g * NTH + _t, _g] = 1.0
        _ETT[_g * NTH + _t, _t] = 1.0
_REPT = np.kron(np.eye(AT, dtype=np.float32), np.ones((1, NBRT), np.float32))
_REPD = np.kron(np.eye(AT, dtype=np.float32), np.ones((1, NBR), np.float32))
_SEGT = _REPT.T.copy()
_SEGD = _REPD.T.copy()


def _ssp(x):
    return jnp.maximum(x, 0.0) + jnp.log(1.0 + jnp.exp(-jnp.abs(x))) - _LOG2


def _cutoff(r):
    return 0.5 * (jnp.cos(r * (np.pi / CUTOFF)) + 1.0) * (r < CUTOFF).astype(r.dtype)


def _dot(a, b):
    return jnp.dot(a, b, preferred_element_type=jnp.float32)


_SC_NC = 2
_SC_NS = 16
_NW = _SC_NC * _SC_NS
_EPW = BA // _NW


def _embed_sc_body(atn_hbm, emb_hbm, out_hbm, idx_v, rows_v, sem):
    wid = lax.axis_index("s") * _SC_NC + lax.axis_index("c")
    base = wid * _EPW
    pltpu.sync_copy(atn_hbm.at[pl.ds(base, _EPW)], idx_v)
    pltpu.async_copy(emb_hbm.at[idx_v], rows_v, sem).wait()
    pltpu.sync_copy(rows_v, out_hbm.at[pl.ds(base, _EPW)])


def _embed_sc(atn_flat, emb):
    k = functools.partial(
        pl.kernel,
        mesh=plsc.VectorSubcoreMesh(core_axis_name="c", subcore_axis_name="s"),
        out_type=jax.ShapeDtypeStruct((BA, F), jnp.float32),
        scratch_types=[
            pltpu.VMEM((_EPW,), jnp.int32),
            pltpu.VMEM((_EPW, F), jnp.float32),
            pltpu.SemaphoreType.DMA,
        ],
    )(_embed_sc_body)
    return k(atn_flat, emb)


def _fused_body(x0_ref, pos_ref, nbrd_ref, nbrj_ref, nbrk_ref,
                nmask_ref, tmask_ref,
                offc_ref, ctc_ref, stc_ref, egt_ref, ett_ref,
                rept_ref, repd_ref, segt_ref, segd_ref,
                fdw1_ref, fdb1_ref, fdw2_ref, fdb2_ref,
                ftw1_ref, ftb1_ref, ftw2_ref, ftb2_ref,
                in2f_ref, f2ow_ref, f2ob_ref, dw_ref, db_ref,
                xo_ref):
    bf16 = jnp.bfloat16
    f32 = jnp.float32
    pos3 = pos_ref[0]
    pos_t = jnp.concatenate(
        [pos3, jnp.zeros((AT, 5), f32)], axis=1).T
    pos_hi = pos_t.astype(bf16)
    pos_lo = (pos_t - pos_hi.astype(f32)).astype(bf16)
    pos_hl = jnp.concatenate([pos_hi, pos_lo], axis=0)
    offc = offc_ref[...]

    io_t = jax.lax.broadcasted_iota(jnp.int32, (AT, RTB), 0)

    def onehot_b(idx_row, width):
        return (io_t[:, :width] == idx_row).astype(bf16)

    def gather_pos(oh_b):
        g = _dot(pos_hl, oh_b)
        return g[:8, :] + g[8:, :]

    bias_cols = jnp.concatenate(
        [fdb1_ref[...], fdb2_ref[...], ftb1_ref[...], ftb2_ref[...],
         f2ob_ref[...], db_ref[...]], axis=0).T

    def bcol(j):
        return bias_cols[:, j:j + 1]

    ohd_b = onehot_b(nbrd_ref[0], RDB)
    pj_d = gather_pos(ohd_b)
    pi_d = gather_pos(repd_ref[...])
    vd = pj_d - pi_d
    rdst = jnp.sqrt(jnp.sum(vd * vd, axis=0, keepdims=True) + 1e-9)
    sdd = rdst - offc
    fd_b = jnp.exp((-0.5 / _W2) * sdd * sdd).astype(bf16)
    cdf = _cutoff(rdst) * nmask_ref[0]

    ohj_b = onehot_b(nbrj_ref[0], RTB)
    ohk_b = onehot_b(nbrk_ref[0], RTB)
    pi_t = gather_pos(rept_ref[...])
    vij = gather_pos(ohj_b) - pi_t
    vik = gather_pos(ohk_b) - pi_t
    rij = jnp.sqrt(jnp.sum(vij * vij, axis=0, keepdims=True) + 1e-9)
    rik = jnp.sqrt(jnp.sum(vik * vik, axis=0, keepdims=True) + 1e-9)
    cost = jnp.sum(vij * vik, axis=0, keepdims=True) / (rij * rik)
    cost = jnp.clip(cost, -1.0 + 1e-6, 1.0 - 1e-6)
    sint = jnp.sqrt(1.0 - cost * cost)
    sij = rij - offc
    sik = rik - offc
    gr_b = jnp.exp((-0.5 / _W2) * (sij * sij + sik * sik)).astype(bf16)
    base = 1.0 + ctc_ref[...] * cost + stc_ref[...] * sint
    b2 = base * base
    b4 = b2 * b2
    ang_b = ((b4 * b4) * (2.0 ** (1.0 - ZETA))).astype(bf16)
    rboth = jnp.concatenate([rij, rik], axis=0)
    cutb = _cutoff(rboth)
    ctf = cutb[0:1, :] * cutb[1:2, :] * tmask_ref[0]

    egt_b = egt_ref[...].astype(bf16)
    ett_b = ett_ref[...].astype(bf16)
    segt_b = segt_ref[...]
    segd_b = segd_ref[...]

    x_t = x0_ref[...].T
    for i in range(NINT):
        fdw1t = fdw1_ref[i].T.astype(bf16)
        fdw2t = fdw2_ref[i].T.astype(bf16)
        ftw1t = ftw1_ref[i].T.astype(bf16)
        ftw2t = ftw2_ref[i].T.astype(bf16)
        in2ft = in2f_ref[i].T
        f2owt = f2ow_ref[i].T
        dwt = dw_ref[i].T
        y_t = _dot(in2ft, x_t)
        y_b = y_t.astype(bf16)
        hd = _ssp(_dot(fdw1t, fd_b) + bcol(i))
        wd = (_dot(fdw2t, hd.astype(bf16)) + bcol(NINT + i)) * cdf
        prod_d = (_dot(y_b, ohd_b) * wd).astype(bf16)
        agg = _dot(prod_d, segd_b)
        for c in range(NCH):
            lo, hi = c * RTC, (c + 1) * RTC
            feat = (_dot(egt_b, gr_b[:, lo:hi])
                    * _dot(ett_b, ang_b[:, lo:hi])).astype(bf16)
            ht = _ssp(_dot(ftw1t, feat) + bcol(2 * NINT + i))
            wt = (_dot(ftw2t, ht.astype(bf16))
                  + bcol(3 * NINT + i)) * ctf[:, lo:hi]
            prod_t = (_dot(y_b, ohj_b[:, lo:hi]) * _dot(y_b, ohk_b[:, lo:hi])
                      * wt).astype(bf16)
            agg = agg + _dot(prod_t, segt_b[lo:hi, :])
        v = _ssp(_dot(f2owt, agg) + bcol(4 * NINT + i))
        x_t = x_t + _dot(dwt, v) + bcol(5 * NINT + i)
    xo_ref[0] = x_t.T


def kernel(atomic_numbers, positions, neighbors, neighbor_mask, neighbors_j,
           neighbors_k, triple_mask, emb, fd_W1, fd_b1, fd_W2, fd_b2,
           ft_W1, ft_b1, ft_W2, ft_b2, in2f_W, f2out_W, f2out_b,
           dense_W, dense_b):
    f32 = jnp.float32
    bf16 = jnp.bfloat16
    nbrd_b = neighbors.astype(jnp.int32).reshape(B, 1, RDB)
    nbrj_b = neighbors_j.astype(jnp.int32).reshape(B, 1, RTB)
    nbrk_b = neighbors_k.astype(jnp.int32).reshape(B, 1, RTB)
    nmask_b = neighbor_mask.astype(f32).reshape(B, 1, RDB)
    tmask_b = triple_mask.astype(f32).reshape(B, 1, RTB)

    x0 = _embed_sc(atomic_numbers.astype(jnp.int32).reshape(BA),
                   emb.astype(f32))

    whole = lambda *shape: pl.BlockSpec(shape, lambda g: tuple(0 for _ in shape))
    perb = lambda *shape: pl.BlockSpec((1,) + shape, lambda g: (g,) + tuple(
        0 for _ in shape))

    out = pl.pallas_call(
        _fused_body,
        grid=(B,),
        in_specs=[
            pl.BlockSpec((AT, F), lambda g: (g, 0)),
            perb(AT, 3),
            perb(1, RDB), perb(1, RTB), perb(1, RTB),
            perb(1, RDB), perb(1, RTB),
            whole(NGT, 1), whole(NTH, 1), whole(NTH, 1),
            whole(FTW, NGT), whole(FTW, NTH),
            whole(AT, RTB), whole(AT, RDB),
            whole(RTB, AT), whole(RDB, AT),
            whole(NINT, NGD, F), whole(NINT, F),
            whole(NINT, F, F), whole(NINT, F),
            whole(NINT, FTW, F), whole(NINT, F),
            whole(NINT, F, F), whole(NINT, F),
            whole(NINT, F, F), whole(NINT, F, F), whole(NINT, F),
            whole(NINT, F, F), whole(NINT, F),
        ],
        out_specs=pl.BlockSpec((1, AT, F), lambda g: (g, 0, 0)),
        out_shape=jax.ShapeDtypeStruct((B, AT, F), f32),
    )(x0, positions, nbrd_b, nbrj_b, nbrk_b, nmask_b, tmask_b,
      jnp.asarray(_OFFCOL), jnp.asarray(_CTCOL), jnp.asarray(_STCOL),
      jnp.asarray(_EGT), jnp.asarray(_ETT),
      jnp.asarray(_REPT).astype(bf16), jnp.asarray(_REPD).astype(bf16),
      jnp.asarray(_SEGT).astype(bf16), jnp.asarray(_SEGD).astype(bf16),
      fd_W1, fd_b1, fd_W2, fd_b2, ft_W1, ft_b1, ft_W2, ft_b2,
      in2f_W, f2out_W, f2out_b, dense_W, dense_b)
    return out
